# baseline (speedup 1.0000x reference)
import functools

import jax
import jax.numpy as jnp
from jax import lax
from jax.experimental import pallas as pl
from jax.experimental.pallas import tpu as pltpu
from jax.experimental.pallas import tpu_sc as plsc

NW = 32
NC = 2
LANES = 16
GROW = 128
WSZ = 45056


def _k1_body(numy, y_ref, w_ref, safe_ref, tail_ref):
    y = y_ref[...]
    safe_ref[...] = jnp.where(y < numy, y, 0)
    tail_ref[...] = w_ref[...]


def _k2_body(nk, nir, rpw, numy, y3_ref, safe_hbm, w_hbm, wtail_hbm,
             buf_ref, wg_ref, yv, tv, sidx, wina, winb, res,
             sem0, sem1, sem2):
    w = lax.axis_index("s") * NC + lax.axis_index("c")
    base_t = w * nk * LANES
    pltpu.sync_copy(y3_ref.at[w], yv)
    for k in range(nk):
        tv[k] = base_t + k * LANES + lax.iota(jnp.int32, LANES)
    ds = [pltpu.async_copy(tv.at[k], buf_ref.at[yv.at[k]], sem0)
          for k in range(nk)]
    for d in ds:
        d.wait()
    pltpu.sync_copy(safe_hbm, sidx)
    numy128 = (numy // GROW) * GROW
    tail = numy - numy128
    wins = []
    for rr in range(rpw):
        for o in range(0, numy128, WSZ):
            wins.append((rr, o, min(WSZ, numy128 - o), False, False))
        if tail:
            wins.append((rr, numy128, tail, True, False))
        wins[-1] = wins[-1][:4] + (True,)
    bufs = (wina, winb)
    sems = (sem1, sem2)

    def issue(i):
        rr, w0, wlen, is_tail, _ = wins[i]
        r = w * rpw + rr
        dst = bufs[i % 2]
        if is_tail:
            return pltpu.async_copy(wtail_hbm.at[pl.ds(r * GROW, GROW)],
                                    dst.at[pl.ds(0, GROW)], sems[i % 2])
        return pltpu.async_copy(w_hbm.at[r, pl.ds(w0, wlen)],
                                dst.at[pl.ds(0, wlen)], sems[i % 2])

    def scan(i):
        rr, w0, wlen, _, _ = wins[i]
        cur = bufs[i % 2]

        def scan_row(j, c):
            jv = jnp.full((LANES,), 0, jnp.int32) + j
            rv = jv + rr * nir
            for b in range(GROW // LANES):
                lanev = b * LANES + lax.iota(jnp.int32, LANES)
                idx = plsc.load_gather(sidx, [jv, lanev])
                m = (idx >= w0) & (idx < w0 + wlen)
                v = plsc.load_gather(cur, [idx - w0], mask=m)
                plsc.store_scatter(res, [rv, lanev], v, mask=m)
            return c

        lax.fori_loop(0, nir, scan_row, 0)

    pend = issue(0)
    for i in range(len(wins)):
        nxt = issue(i + 1) if i + 1 < len(wins) else None
        pend.wait()
        pend = nxt
        scan(i)
        rr, _, _, _, last_of_row = wins[i]
        if last_of_row:
            r = w * rpw + rr
            pltpu.sync_copy(res.at[pl.ds(rr * nir, nir)], wg_ref.at[r])


def _k3_body(nk, numy, y3_ref, buf_hbm, gf_ref, yv, win, gout, sem0):
    w = lax.axis_index("s") * NC + lax.axis_index("c")
    base_t = w * nk * LANES
    pltpu.sync_copy(y3_ref.at[w], yv)
    ds = [pltpu.async_copy(buf_hbm.at[yv.at[k]], win.at[k], sem0)
          for k in range(nk)]
    for d in ds:
        d.wait()
    for k in range(nk):
        t = base_t + k * LANES + lax.iota(jnp.int32, LANES)
        g = (win[k] == t) & (yv[k] < numy)
        gout[k] = jnp.where(g, 1.0, 0.0).astype(jnp.float32)
    pltpu.sync_copy(gout, gf_ref.at[w])


def _k4_body(numy, P, nsteps, rows_per_blk,
             x_ref, xr_ref, wg_ref, gf_ref, y_ref, out_ref, acc_ref):
    i = pl.program_id(0)

    @pl.when(i == 0)
    def _():
        acc_ref[0] = 0.0
        acc_ref[1] = 0.0

    xv = x_ref[...]
    wg = wg_ref[...]
    gf = gf_ref[0, 0, :]
    bc = wg.shape[1]
    nt = (((1,), (0,)), ((), ()))
    l = lax.dot_general(xv, wg, nt,
                        preferred_element_type=jnp.float32)
    sp = jnp.maximum(l, 0.0) + jnp.log1p(jnp.exp(-jnp.abs(l)))
    t1 = jnp.sum(jnp.sum(sp, axis=0) * gf)

    xb = xr_ref[...]
    l2 = lax.dot_general(xb, wg, nt,
                         preferred_element_type=jnp.float32)
    y = y_ref[...]
    cols = [y[:, k:k + 1] for k in range(P)]
    rf_cols = []
    for p_ in range(P):
        dup = jnp.zeros_like(cols[p_], dtype=jnp.bool_)
        for q_ in range(p_):
            dup = dup | (cols[p_] == cols[q_])
        keep = (cols[p_] < numy) & (~dup)
        rf_cols.append(jnp.where(keep, 1.0, 0.0).astype(jnp.float32))
    rf = jnp.concatenate(rf_cols, axis=1)
    pj = lax.broadcasted_iota(jnp.int32, (P, bc), 1)
    pi = lax.broadcasted_iota(jnp.int32, (P, bc), 0)
    pmat = jnp.where(pj % P == pi, 1.0, 0.0).astype(jnp.float32)
    rf_exp = lax.dot_general(rf, pmat, (((1,), (0,)), ((), ())),
                             preferred_element_type=jnp.float32)
    bj = lax.broadcasted_iota(jnp.int32, (rows_per_blk, bc), 1)
    bi = lax.broadcasted_iota(jnp.int32, (rows_per_blk, bc), 0)
    bd = jnp.where(bj // P == bi, 1.0, 0.0).astype(jnp.float32)
    t2 = jnp.sum(l2 * rf_exp * bd)

    acc_ref[0] += t1 - t2
    acc_ref[1] += jnp.sum(gf)

    @pl.when(i == nsteps - 1)
    def _():
        out_ref[...] = (acc_ref[0] / (xv.shape[0] * acc_ref[1])).reshape(1, 1)


def kernel(x, W, y_inds, numy):
    B, d = x.shape
    P = y_inds.shape[1]
    numy_s = W.shape[1]
    K = B * P
    chunk = K // NW
    nk = chunk // LANES
    nir = K // GROW
    rpw = d // NW

    yflat = y_inds.reshape(K)
    y2d = yflat.reshape(nir, GROW)
    y3 = yflat.reshape(NW, nk, LANES)

    ntile = numy_s // GROW
    safe2d, wtail2d = pl.pallas_call(
        functools.partial(_k1_body, numy_s),
        grid=(1,),
        in_specs=[
            pl.BlockSpec((nir, GROW), lambda i: (0, 0)),
            pl.BlockSpec((d, GROW), lambda i: (0, ntile)),
        ],
        out_specs=[
            pl.BlockSpec((nir, GROW), lambda i: (0, 0)),
            pl.BlockSpec((d, GROW), lambda i: (0, 0)),
        ],
        out_shape=[
            jax.ShapeDtypeStruct((nir, GROW), jnp.int32),
            jax.ShapeDtypeStruct((d, GROW), jnp.float32),
        ],
    )(y2d, W)
    wtail = wtail2d.reshape(d * GROW)

    mesh = plsc.VectorSubcoreMesh(core_axis_name="c", subcore_axis_name="s")
    buf, wg = pl.kernel(
        functools.partial(_k2_body, nk, nir, rpw, numy_s),
        out_type=(
            jax.ShapeDtypeStruct((numy_s + LANES,), jnp.int32),
            jax.ShapeDtypeStruct((d, nir, GROW), jnp.float32),
        ),
        mesh=mesh,
        compiler_params=pltpu.CompilerParams(needs_layout_passes=False),
        scratch_types=[
            pltpu.VMEM((nk, LANES), jnp.int32),
            pltpu.VMEM((nk, LANES), jnp.int32),
            pltpu.VMEM((nir, GROW), jnp.int32),
            pltpu.VMEM((WSZ,), jnp.float32),
            pltpu.VMEM((WSZ,), jnp.float32),
            pltpu.VMEM((rpw * nir, GROW), jnp.float32),
            pltpu.SemaphoreType.DMA,
            pltpu.SemaphoreType.DMA,
            pltpu.SemaphoreType.DMA,
        ],
    )(y3, safe2d, W, wtail)

    gf3 = pl.kernel(
        functools.partial(_k3_body, nk, numy_s),
        out_type=jax.ShapeDtypeStruct((NW, nk, LANES), jnp.float32),
        mesh=mesh,
        scratch_types=[
            pltpu.VMEM((nk, LANES), jnp.int32),
            pltpu.VMEM((nk, LANES), jnp.int32),
            pltpu.VMEM((nk, LANES), jnp.float32),
            pltpu.SemaphoreType.DMA,
        ],
    )(y3, buf)

    nsteps = 8
    bc = K // nsteps
    rpb = bc // P
    gf_r = gf3.reshape(nsteps, 1, bc)
    wg2 = wg.reshape(d, K)
    loss = pl.pallas_call(
        functools.partial(_k4_body, numy_s, P, nsteps, rpb),
        grid=(nsteps,),
        in_specs=[
            pl.BlockSpec((B, d), lambda i: (0, 0)),
            pl.BlockSpec((rpb, d), lambda i: (i, 0)),
            pl.BlockSpec((d, bc), lambda i: (0, i)),
            pl.BlockSpec((1, 1, bc), lambda i: (i, 0, 0)),
            pl.BlockSpec((rpb, P), lambda i: (i, 0)),
        ],
        out_specs=pl.BlockSpec((1, 1), lambda i: (0, 0)),
        out_shape=jax.ShapeDtypeStruct((1, 1), jnp.float32),
        scratch_shapes=[pltpu.SMEM((2,), jnp.float32)],
    )(x, x, wg2, gf_r, y_inds)

    return loss.reshape(())


# speedup vs baseline: 1.0982x; 1.0982x over previous
import functools

import jax
import jax.numpy as jnp
from jax import lax
from jax.experimental import pallas as pl
from jax.experimental.pallas import tpu as pltpu
from jax.experimental.pallas import tpu_sc as plsc

NW = 32
NC = 2
LANES = 16
GROW = 128
WSZ = 47616


def _k1_body(numy, y_ref, w_ref, safe_ref, tail_ref):
    y = y_ref[...]
    safe_ref[...] = jnp.where(y < numy, y, 0)
    tail_ref[...] = w_ref[...]


def _k2_body(nk, nir, rpw, numy, y3_ref, safe_hbm, w_hbm, wtail_hbm,
             buf_ref, wg_ref, yv, tv, sidx, wina, winb, res,
             sem0, sem1, sem2):
    w = lax.axis_index("s") * NC + lax.axis_index("c")
    base_t = w * nk * LANES
    pltpu.sync_copy(y3_ref.at[w], yv)
    for k in range(nk):
        tv[k] = base_t + k * LANES + lax.iota(jnp.int32, LANES)
    ds = [pltpu.async_copy(tv.at[k], buf_ref.at[yv.at[k]], sem0)
          for k in range(nk)]
    for d in ds:
        d.wait()
    pltpu.sync_copy(safe_hbm, sidx)
    numy128 = (numy // GROW) * GROW
    tail = numy - numy128
    wins = []
    for rr in range(rpw):
        for o in range(0, numy128, WSZ):
            wins.append((rr, o, min(WSZ, numy128 - o), False, False))
        if tail:
            wins.append((rr, numy128, tail, True, False))
        wins[-1] = wins[-1][:4] + (True,)
    bufs = (wina, winb)
    sems = (sem1, sem2)

    def issue(i):
        rr, w0, wlen, is_tail, _ = wins[i]
        r = w * rpw + rr
        dst = bufs[i % 2]
        if is_tail:
            return pltpu.async_copy(wtail_hbm.at[pl.ds(r * GROW, GROW)],
                                    dst.at[pl.ds(0, GROW)], sems[i % 2])
        return pltpu.async_copy(w_hbm.at[r, pl.ds(w0, wlen)],
                                dst.at[pl.ds(0, wlen)], sems[i % 2])

    def scan(i):
        _, w0, wlen, _, _ = wins[i]
        cur = bufs[i % 2]

        def scan_row(j, addr):
            for b in range(GROW // LANES):
                ab = addr + b * LANES
                idx = plsc.load_gather(sidx, [ab])
                d = idx - w0
                m = (d >= 0) & (d < wlen)
                v = plsc.load_gather(cur, [d], mask=m)
                plsc.store_scatter(res, [ab], v, mask=m)
            return addr + GROW

        lax.fori_loop(0, nir, scan_row, lax.iota(jnp.int32, LANES))

    pend = issue(0)
    for i in range(len(wins)):
        nxt = issue(i + 1) if i + 1 < len(wins) else None
        pend.wait()
        pend = nxt
        scan(i)
        rr, _, _, _, last_of_row = wins[i]
        if last_of_row:
            r = w * rpw + rr
            pltpu.sync_copy(res, wg_ref.at[r])


def _k3_body(nk, numy, y3_ref, buf_hbm, gf_ref, yv, win, gout, sem0):
    w = lax.axis_index("s") * NC + lax.axis_index("c")
    base_t = w * nk * LANES
    pltpu.sync_copy(y3_ref.at[w], yv)
    ds = [pltpu.async_copy(buf_hbm.at[yv.at[k]], win.at[k], sem0)
          for k in range(nk)]
    for d in ds:
        d.wait()
    for k in range(nk):
        t = base_t + k * LANES + lax.iota(jnp.int32, LANES)
        g = (win[k] == t) & (yv[k] < numy)
        gout[k] = jnp.where(g, 1.0, 0.0).astype(jnp.float32)
    pltpu.sync_copy(gout, gf_ref.at[w])


def _k4_body(numy, P, nsteps, rows_per_blk,
             x_ref, xr_ref, wg_ref, gf_ref, y_ref, out_ref, acc_ref):
    i = pl.program_id(0)

    @pl.when(i == 0)
    def _():
        acc_ref[0] = 0.0
        acc_ref[1] = 0.0

    xv = x_ref[...]
    wg = wg_ref[...]
    gf = gf_ref[0, 0, :]
    bc = wg.shape[1]
    nt = (((1,), (0,)), ((), ()))
    l = lax.dot_general(xv, wg, nt,
                        preferred_element_type=jnp.float32)
    sp = jnp.maximum(l, 0.0) + jnp.log1p(jnp.exp(-jnp.abs(l)))
    t1 = jnp.sum(jnp.sum(sp, axis=0) * gf)

    xb = xr_ref[...]
    l2 = lax.dot_general(xb, wg, nt,
                         preferred_element_type=jnp.float32)
    y = y_ref[...]
    cols = [y[:, k:k + 1] for k in range(P)]
    rf_cols = []
    for p_ in range(P):
        dup = jnp.zeros_like(cols[p_], dtype=jnp.bool_)
        for q_ in range(p_):
            dup = dup | (cols[p_] == cols[q_])
        keep = (cols[p_] < numy) & (~dup)
        rf_cols.append(jnp.where(keep, 1.0, 0.0).astype(jnp.float32))
    rf = jnp.concatenate(rf_cols, axis=1)
    pj = lax.broadcasted_iota(jnp.int32, (P, bc), 1)
    pi = lax.broadcasted_iota(jnp.int32, (P, bc), 0)
    pmat = jnp.where(pj % P == pi, 1.0, 0.0).astype(jnp.float32)
    rf_exp = lax.dot_general(rf, pmat, (((1,), (0,)), ((), ())),
                             preferred_element_type=jnp.float32)
    bj = lax.broadcasted_iota(jnp.int32, (rows_per_blk, bc), 1)
    bi = lax.broadcasted_iota(jnp.int32, (rows_per_blk, bc), 0)
    bd = jnp.where(bj // P == bi, 1.0, 0.0).astype(jnp.float32)
    t2 = jnp.sum(l2 * rf_exp * bd)

    acc_ref[0] += t1 - t2
    acc_ref[1] += jnp.sum(gf)

    @pl.when(i == nsteps - 1)
    def _():
        out_ref[...] = (acc_ref[0] / (xv.shape[0] * acc_ref[1])).reshape(1, 1)


def kernel(x, W, y_inds, numy):
    B, d = x.shape
    P = y_inds.shape[1]
    numy_s = W.shape[1]
    K = B * P
    chunk = K // NW
    nk = chunk // LANES
    nir = K // GROW
    rpw = d // NW

    yflat = y_inds.reshape(K)
    y2d = yflat.reshape(nir, GROW)
    y3 = yflat.reshape(NW, nk, LANES)

    ntile = numy_s // GROW
    safe2d, wtail2d = pl.pallas_call(
        functools.partial(_k1_body, numy_s),
        grid=(1,),
        in_specs=[
            pl.BlockSpec((nir, GROW), lambda i: (0, 0)),
            pl.BlockSpec((d, GROW), lambda i: (0, ntile)),
        ],
        out_specs=[
            pl.BlockSpec((nir, GROW), lambda i: (0, 0)),
            pl.BlockSpec((d, GROW), lambda i: (0, 0)),
        ],
        out_shape=[
            jax.ShapeDtypeStruct((nir, GROW), jnp.int32),
            jax.ShapeDtypeStruct((d, GROW), jnp.float32),
        ],
    )(y2d, W)
    wtail = wtail2d.reshape(d * GROW)

    mesh = plsc.VectorSubcoreMesh(core_axis_name="c", subcore_axis_name="s")
    buf, wg = pl.kernel(
        functools.partial(_k2_body, nk, nir, rpw, numy_s),
        out_type=(
            jax.ShapeDtypeStruct((numy_s + LANES,), jnp.int32),
            jax.ShapeDtypeStruct((d, K), jnp.float32),
        ),
        mesh=mesh,
        compiler_params=pltpu.CompilerParams(needs_layout_passes=False),
        scratch_types=[
            pltpu.VMEM((nk, LANES), jnp.int32),
            pltpu.VMEM((nk, LANES), jnp.int32),
            pltpu.VMEM((K,), jnp.int32),
            pltpu.VMEM((WSZ,), jnp.float32),
            pltpu.VMEM((WSZ,), jnp.float32),
            pltpu.VMEM((K,), jnp.float32),
            pltpu.SemaphoreType.DMA,
            pltpu.SemaphoreType.DMA,
            pltpu.SemaphoreType.DMA,
        ],
    )(y3, safe2d.reshape(K), W, wtail)

    gf3 = pl.kernel(
        functools.partial(_k3_body, nk, numy_s),
        out_type=jax.ShapeDtypeStruct((NW, nk, LANES), jnp.float32),
        mesh=mesh,
        scratch_types=[
            pltpu.VMEM((nk, LANES), jnp.int32),
            pltpu.VMEM((nk, LANES), jnp.int32),
            pltpu.VMEM((nk, LANES), jnp.float32),
            pltpu.SemaphoreType.DMA,
        ],
    )(y3, buf)

    nsteps = 8
    bc = K // nsteps
    rpb = bc // P
    gf_r = gf3.reshape(nsteps, 1, bc)
    wg2 = wg
    loss = pl.pallas_call(
        functools.partial(_k4_body, numy_s, P, nsteps, rpb),
        grid=(nsteps,),
        in_specs=[
            pl.BlockSpec((B, d), lambda i: (0, 0)),
            pl.BlockSpec((rpb, d), lambda i: (i, 0)),
            pl.BlockSpec((d, bc), lambda i: (0, i)),
            pl.BlockSpec((1, 1, bc), lambda i: (i, 0, 0)),
            pl.BlockSpec((rpb, P), lambda i: (i, 0)),
        ],
        out_specs=pl.BlockSpec((1, 1), lambda i: (0, 0)),
        out_shape=jax.ShapeDtypeStruct((1, 1), jnp.float32),
        scratch_shapes=[pltpu.SMEM((2,), jnp.float32)],
    )(x, x, wg2, gf_r, y_inds)

    return loss.reshape(())


# speedup vs baseline: 1.1032x; 1.0046x over previous
import functools

import jax
import jax.numpy as jnp
from jax import lax
from jax.experimental import pallas as pl
from jax.experimental.pallas import tpu as pltpu
from jax.experimental.pallas import tpu_sc as plsc

NW = 32
NC = 2
LANES = 16
GROW = 128
WSZ = 47616


def _k1_body(numy, y_ref, w_ref, safe_ref, tail_ref):
    y = y_ref[...]
    safe_ref[...] = jnp.where(y < numy, y, 0)
    tail_ref[...] = w_ref[...]


def _k2_body(nk, nir, rpw, numy, y3_ref, safe_hbm, w_hbm, wtail_hbm,
             buf_ref, wg_ref, yv, tv, sidx, wina, winb, res,
             sem0, sem1, sem2):
    w = lax.axis_index("s") * NC + lax.axis_index("c")
    base_t = w * nk * LANES
    pltpu.sync_copy(y3_ref.at[w], yv)
    for k in range(nk):
        tv[k] = base_t + k * LANES + lax.iota(jnp.int32, LANES)
    ds = [pltpu.async_copy(tv.at[k], buf_ref.at[yv.at[k]], sem0)
          for k in range(nk)]
    for d in ds:
        d.wait()
    pltpu.sync_copy(safe_hbm, sidx)
    numy128 = (numy // GROW) * GROW
    tail = numy - numy128
    wins = []
    for rr in range(rpw):
        for o in range(0, numy128, WSZ):
            wins.append((rr, o, min(WSZ, numy128 - o), False, False))
        if tail:
            wins.append((rr, numy128, tail, True, False))
        wins[-1] = wins[-1][:4] + (True,)
    bufs = (wina, winb)
    sems = (sem1, sem2)

    def issue(i):
        rr, w0, wlen, is_tail, _ = wins[i]
        r = w * rpw + rr
        dst = bufs[i % 2]
        if is_tail:
            return pltpu.async_copy(wtail_hbm.at[pl.ds(r * GROW, GROW)],
                                    dst.at[pl.ds(0, GROW)], sems[i % 2])
        return pltpu.async_copy(w_hbm.at[r, pl.ds(w0, wlen)],
                                dst.at[pl.ds(0, wlen)], sems[i % 2])

    def scan(i):
        _, w0, wlen, _, _ = wins[i]
        cur = bufs[i % 2]
        half = nir // 2

        def scan_row(j, addr):
            for h in range(2):
                a0 = addr + h * (half * GROW)
                for b in range(GROW // LANES):
                    ab = a0 + b * LANES
                    idx = plsc.load_gather(sidx, [ab])
                    d = idx - w0
                    m = d.astype(jnp.uint32) < jnp.uint32(wlen)
                    v = plsc.load_gather(cur, [d], mask=m)
                    plsc.store_scatter(res, [ab], v, mask=m)
            return addr + GROW

        lax.fori_loop(0, half, scan_row, lax.iota(jnp.int32, LANES))

    pend = issue(0)
    for i in range(len(wins)):
        nxt = issue(i + 1) if i + 1 < len(wins) else None
        pend.wait()
        pend = nxt
        scan(i)
        rr, _, _, _, last_of_row = wins[i]
        if last_of_row:
            r = w * rpw + rr
            pltpu.sync_copy(res, wg_ref.at[r])


def _k3_body(nk, numy, y3_ref, buf_hbm, gf_ref, yv, win, gout, sem0):
    w = lax.axis_index("s") * NC + lax.axis_index("c")
    base_t = w * nk * LANES
    pltpu.sync_copy(y3_ref.at[w], yv)
    ds = [pltpu.async_copy(buf_hbm.at[yv.at[k]], win.at[k], sem0)
          for k in range(nk)]
    for d in ds:
        d.wait()
    for k in range(nk):
        t = base_t + k * LANES + lax.iota(jnp.int32, LANES)
        g = (win[k] == t) & (yv[k] < numy)
        gout[k] = jnp.where(g, 1.0, 0.0).astype(jnp.float32)
    pltpu.sync_copy(gout, gf_ref.at[w])


def _k4_body(numy, P, nsteps, rows_per_blk,
             x_ref, xr_ref, wg_ref, gf_ref, y_ref, out_ref, acc_ref):
    i = pl.program_id(0)

    @pl.when(i == 0)
    def _():
        acc_ref[0] = 0.0
        acc_ref[1] = 0.0

    xv = x_ref[...]
    wg = wg_ref[...]
    gf = gf_ref[0, 0, :]
    bc = wg.shape[1]
    nt = (((1,), (0,)), ((), ()))
    l = lax.dot_general(xv, wg, nt,
                        preferred_element_type=jnp.float32)
    sp = jnp.maximum(l, 0.0) + jnp.log1p(jnp.exp(-jnp.abs(l)))
    t1 = jnp.sum(jnp.sum(sp, axis=0) * gf)

    xb = xr_ref[...]
    l2 = lax.dot_general(xb, wg, nt,
                         preferred_element_type=jnp.float32)
    y = y_ref[...]
    cols = [y[:, k:k + 1] for k in range(P)]
    rf_cols = []
    for p_ in range(P):
        dup = jnp.zeros_like(cols[p_], dtype=jnp.bool_)
        for q_ in range(p_):
            dup = dup | (cols[p_] == cols[q_])
        keep = (cols[p_] < numy) & (~dup)
        rf_cols.append(jnp.where(keep, 1.0, 0.0).astype(jnp.float32))
    rf = jnp.concatenate(rf_cols, axis=1)
    pj = lax.broadcasted_iota(jnp.int32, (P, bc), 1)
    pi = lax.broadcasted_iota(jnp.int32, (P, bc), 0)
    pmat = jnp.where(pj % P == pi, 1.0, 0.0).astype(jnp.float32)
    rf_exp = lax.dot_general(rf, pmat, (((1,), (0,)), ((), ())),
                             preferred_element_type=jnp.float32)
    bj = lax.broadcasted_iota(jnp.int32, (rows_per_blk, bc), 1)
    bi = lax.broadcasted_iota(jnp.int32, (rows_per_blk, bc), 0)
    bd = jnp.where(bj // P == bi, 1.0, 0.0).astype(jnp.float32)
    t2 = jnp.sum(l2 * rf_exp * bd)

    acc_ref[0] += t1 - t2
    acc_ref[1] += jnp.sum(gf)

    @pl.when(i == nsteps - 1)
    def _():
        out_ref[...] = (acc_ref[0] / (xv.shape[0] * acc_ref[1])).reshape(1, 1)


def kernel(x, W, y_inds, numy):
    B, d = x.shape
    P = y_inds.shape[1]
    numy_s = W.shape[1]
    K = B * P
    chunk = K // NW
    nk = chunk // LANES
    nir = K // GROW
    rpw = d // NW

    yflat = y_inds.reshape(K)
    y2d = yflat.reshape(nir, GROW)
    y3 = yflat.reshape(NW, nk, LANES)

    ntile = numy_s // GROW
    safe2d, wtail2d = pl.pallas_call(
        functools.partial(_k1_body, numy_s),
        grid=(1,),
        in_specs=[
            pl.BlockSpec((nir, GROW), lambda i: (0, 0)),
            pl.BlockSpec((d, GROW), lambda i: (0, ntile)),
        ],
        out_specs=[
            pl.BlockSpec((nir, GROW), lambda i: (0, 0)),
            pl.BlockSpec((d, GROW), lambda i: (0, 0)),
        ],
        out_shape=[
            jax.ShapeDtypeStruct((nir, GROW), jnp.int32),
            jax.ShapeDtypeStruct((d, GROW), jnp.float32),
        ],
    )(y2d, W)
    wtail = wtail2d.reshape(d * GROW)

    mesh = plsc.VectorSubcoreMesh(core_axis_name="c", subcore_axis_name="s")
    buf, wg = pl.kernel(
        functools.partial(_k2_body, nk, nir, rpw, numy_s),
        out_type=(
            jax.ShapeDtypeStruct((numy_s + LANES,), jnp.int32),
            jax.ShapeDtypeStruct((d, K), jnp.float32),
        ),
        mesh=mesh,
        compiler_params=pltpu.CompilerParams(needs_layout_passes=False),
        scratch_types=[
            pltpu.VMEM((nk, LANES), jnp.int32),
            pltpu.VMEM((nk, LANES), jnp.int32),
            pltpu.VMEM((K,), jnp.int32),
            pltpu.VMEM((WSZ,), jnp.float32),
            pltpu.VMEM((WSZ,), jnp.float32),
            pltpu.VMEM((K,), jnp.float32),
            pltpu.SemaphoreType.DMA,
            pltpu.SemaphoreType.DMA,
            pltpu.SemaphoreType.DMA,
        ],
    )(y3, safe2d.reshape(K), W, wtail)

    gf3 = pl.kernel(
        functools.partial(_k3_body, nk, numy_s),
        out_type=jax.ShapeDtypeStruct((NW, nk, LANES), jnp.float32),
        mesh=mesh,
        scratch_types=[
            pltpu.VMEM((nk, LANES), jnp.int32),
            pltpu.VMEM((nk, LANES), jnp.int32),
            pltpu.VMEM((nk, LANES), jnp.float32),
            pltpu.SemaphoreType.DMA,
        ],
    )(y3, buf)

    nsteps = 8
    bc = K // nsteps
    rpb = bc // P
    gf_r = gf3.reshape(nsteps, 1, bc)
    wg2 = wg
    loss = pl.pallas_call(
        functools.partial(_k4_body, numy_s, P, nsteps, rpb),
        grid=(nsteps,),
        in_specs=[
            pl.BlockSpec((B, d), lambda i: (0, 0)),
            pl.BlockSpec((rpb, d), lambda i: (i, 0)),
            pl.BlockSpec((d, bc), lambda i: (0, i)),
            pl.BlockSpec((1, 1, bc), lambda i: (i, 0, 0)),
            pl.BlockSpec((rpb, P), lambda i: (i, 0)),
        ],
        out_specs=pl.BlockSpec((1, 1), lambda i: (0, 0)),
        out_shape=jax.ShapeDtypeStruct((1, 1), jnp.float32),
        scratch_shapes=[pltpu.SMEM((2,), jnp.float32)],
    )(x, x, wg2, gf_r, y_inds)

    return loss.reshape(())
